# Initial kernel scaffold; baseline (speedup 1.0000x reference)
import jax
import jax.numpy as jnp
from jax.experimental import pallas as pl


def kernel(x, edge_index, edge_attr, batch, W1e, b1e, W1, b1, W2e, b2e, W2, b2, W3e, b3e, W3, b3, Wo, bo):
    raise NotImplementedError("write your pallas kernel here")


# speedup vs baseline: 3.6049x; 3.6049x over previous
import jax
import jax.numpy as jnp
from jax import lax
from jax.experimental import pallas as pl
from jax.experimental.pallas import tpu as pltpu
from jax.experimental.pallas import tpu_sc as plsc

N = 10000
E = 640000
NG = 64
H = 128
D1 = 16
NC = 2
NS = 16
NW = NC * NS
C = 128
K = 160
EW = C * K
E_PAD = EW * NW
N_PAD = 10240
RPT = N_PAD // NS
BLK = 2000
G = N // BLK


def _make_edge_phase(d):
    mesh = plsc.VectorSubcoreMesh(core_axis_name="c", subcore_axis_name="s")

    def body(x_hbm, idx_hbm, attr_hbm, evec_hbm, ebias_hbm, zeros_hbm,
             out_hbm, ec_v, attr_c, rows_v, evec_v, ebias_v, agg_sh, sem):
        c = lax.axis_index("c")
        s = lax.axis_index("s")
        base = (c * NS + s) * K
        pltpu.sync_copy(evec_hbm, evec_v)
        pltpu.sync_copy(ebias_hbm, ebias_v)

        @pl.when(s == 0)
        def _():
            pltpu.sync_copy(zeros_hbm, agg_sh)

        plsc.subcore_barrier()

        ev = [evec_v[pl.ds(16 * j, 16)] for j in range(d // 16)]
        eb = [ebias_v[pl.ds(16 * j, 16)] for j in range(d // 16)]

        def chunk_body(k, carry):
            pltpu.sync_copy(idx_hbm.at[base + k], ec_v)
            pltpu.sync_copy(attr_hbm.at[base + k], attr_c)
            pltpu.async_copy(x_hbm.at[ec_v.at[0]], rows_v, sem).wait()

            def group_body(g, carry2):
                a16 = attr_c[pl.ds(g * 16, 16)]
                i0 = g * 16
                for i2 in range(16):
                    a = a16[i2]
                    for j in range(d // 16):
                        sl = pl.ds(16 * j, 16)
                        rows_v[i0 + i2, sl] = jnp.maximum(
                            rows_v[i0 + i2, sl] + eb[j] + a * ev[j], 0.0)
                return carry2

            lax.fori_loop(0, C // 16, group_body, 0)
            pltpu.sync_copy(rows_v, agg_sh.at[ec_v.at[1]], add=True)
            return carry

        lax.fori_loop(0, K, chunk_body, 0)
        plsc.subcore_barrier()
        r0 = s * RPT
        pltpu.sync_copy(agg_sh.at[pl.ds(r0, RPT)],
                        out_hbm.at[c, pl.ds(r0, RPT)])

    return pl.kernel(
        body,
        out_type=jax.ShapeDtypeStruct((NC, N_PAD, d), jnp.float32),
        mesh=mesh,
        compiler_params=pltpu.CompilerParams(use_tc_tiling_on_sc=False),
        scratch_types=[
            pltpu.VMEM((2, C), jnp.int32),
            pltpu.VMEM((C,), jnp.float32),
            pltpu.VMEM((C, d), jnp.float32),
            pltpu.VMEM((d,), jnp.float32),
            pltpu.VMEM((d,), jnp.float32),
            pltpu.VMEM_SHARED((N_PAD, d), jnp.float32),
            pltpu.SemaphoreType.DMA,
        ],
    )


_edge16 = _make_edge_phase(D1)
_edge128 = _make_edge_phase(H)


def _tc_layer(din):
    def body(x_ref, agg_ref, w_ref, b_ref, o_ref):
        h = x_ref[...] + agg_ref[0] + agg_ref[1]
        y = lax.dot_general(h, w_ref[...], (((1,), (1,)), ((), ())),
                            preferred_element_type=jnp.float32)
        o_ref[...] = jnp.maximum(y + b_ref[...], 0.0)

    return pl.pallas_call(
        body,
        grid=(G,),
        in_specs=[
            pl.BlockSpec((BLK, din), lambda i: (i, 0)),
            pl.BlockSpec((NC, BLK, din), lambda i: (0, i, 0)),
            pl.BlockSpec((H, din), lambda i: (0, 0)),
            pl.BlockSpec((1, H), lambda i: (0, 0)),
        ],
        out_specs=pl.BlockSpec((BLK, H), lambda i: (i, 0)),
        out_shape=jax.ShapeDtypeStruct((N, H), jnp.float32),
    )


_layer16 = _tc_layer(D1)
_layer128 = _tc_layer(H)


def _tc_final_body(x_ref, agg_ref, w_ref, b_ref, batch_ref, wo_ref, bo_ref,
                   o_ref, sums, counts):
    i = pl.program_id(0)

    @pl.when(i == 0)
    def _():
        sums[...] = jnp.zeros_like(sums)
        counts[...] = jnp.zeros_like(counts)

    h = x_ref[...] + agg_ref[0] + agg_ref[1]
    y = lax.dot_general(h, w_ref[...], (((1,), (1,)), ((), ())),
                        preferred_element_type=jnp.float32)
    y = jnp.maximum(y + b_ref[...], 0.0)
    bt = batch_ref[0]
    ohT = (lax.broadcasted_iota(jnp.int32, (NG, BLK), 0) == bt)
    ohT = ohT.astype(jnp.float32)
    sums[...] += lax.dot_general(ohT, y, (((1,), (0,)), ((), ())),
                                 preferred_element_type=jnp.float32)
    counts[...] += jnp.sum(ohT, axis=1, keepdims=True)

    @pl.when(i == G - 1)
    def _():
        pooled = sums[...] / jnp.maximum(counts[...], 1.0)
        o_ref[...] = lax.dot_general(pooled, wo_ref[...],
                                     (((1,), (1,)), ((), ())),
                                     preferred_element_type=jnp.float32) \
            + bo_ref[...]


_final = pl.pallas_call(
    _tc_final_body,
    grid=(G,),
    in_specs=[
        pl.BlockSpec((BLK, H), lambda i: (i, 0)),
        pl.BlockSpec((NC, BLK, H), lambda i: (0, i, 0)),
        pl.BlockSpec((H, H), lambda i: (0, 0)),
        pl.BlockSpec((1, H), lambda i: (0, 0)),
        pl.BlockSpec((1, 1, BLK), lambda i: (i, 0, 0)),
        pl.BlockSpec((5, H), lambda i: (0, 0)),
        pl.BlockSpec((1, 5), lambda i: (0, 0)),
    ],
    out_specs=pl.BlockSpec((NG, 5), lambda i: (0, 0)),
    out_shape=jax.ShapeDtypeStruct((NG, 5), jnp.float32),
    scratch_shapes=[
        pltpu.VMEM((NG, H), jnp.float32),
        pltpu.VMEM((NG, 1), jnp.float32),
    ],
)


def kernel(x, edge_index, edge_attr, batch,
           W1e, b1e, W1, b1, W2e, b2e, W2, b2, W3e, b3e, W3, b3, Wo, bo):
    src = edge_index[0]
    dst = edge_index[1]
    attr = edge_attr[:, 0]
    pad = E_PAD - E
    srcs = jnp.concatenate([src, jnp.zeros((pad,), jnp.int32)])
    dsts = jnp.concatenate([dst, jnp.full((pad,), N, jnp.int32)])
    attrs = jnp.concatenate([attr, jnp.zeros((pad,), jnp.float32)])
    idxs = jnp.stack([
        srcs.reshape(E_PAD // C, C),
        dsts.reshape(E_PAD // C, C),
    ], axis=1)
    attrs = attrs.reshape(E_PAD // C, C)

    x1p = jnp.pad(x, ((0, 0), (0, D1 - 7)))
    w1p = jnp.pad(W1, ((0, 0), (0, D1 - 7)))
    ev1 = jnp.pad(W1e[:, 0], (0, D1 - 7))
    eb1 = jnp.pad(b1e, (0, D1 - 7))
    z16 = jnp.zeros((N_PAD, D1), jnp.float32)
    z128 = jnp.zeros((N_PAD, H), jnp.float32)

    agg1 = _edge16(x1p, idxs, attrs, ev1, eb1, z16)
    x2 = _layer16(x1p, agg1, w1p, b1.reshape(1, H))
    agg2 = _edge128(x2, idxs, attrs, W2e[:, 0], b2e, z128)
    x3 = _layer128(x2, agg2, W2, b2.reshape(1, H))
    agg3 = _edge128(x3, idxs, attrs, W3e[:, 0], b3e, z128)
    out = _final(x3, agg3, W3, b3.reshape(1, H),
                 batch.reshape(G, 1, BLK), Wo, bo.reshape(1, 5))
    return out


# speedup vs baseline: 5.3199x; 1.4758x over previous
import jax
import jax.numpy as jnp
from jax import lax
from jax.experimental import pallas as pl
from jax.experimental.pallas import tpu as pltpu
from jax.experimental.pallas import tpu_sc as plsc

N = 10000
E = 640000
NG = 64
H = 128
D1 = 16
NC = 2
NS = 16
NW = NC * NS
C = 128
K = 160
EW = C * K
E_PAD = EW * NW
N_PAD = 10240
RPT = N_PAD // NS
BLK = 2000
G = N // BLK


def _make_edge_phase(d):
    mesh = plsc.VectorSubcoreMesh(core_axis_name="c", subcore_axis_name="s")

    def body(x_hbm, idx_hbm, attr_hbm, evec_hbm, ebias_hbm, zeros_hbm,
             out_hbm, ec_v, attr_c, rows_v, evec_v, ebias_v, agg_sh, sem):
        c = lax.axis_index("c")
        s = lax.axis_index("s")
        base = (c * NS + s) * K
        pltpu.sync_copy(evec_hbm, evec_v)
        pltpu.sync_copy(ebias_hbm, ebias_v)

        @pl.when(s == 0)
        def _():
            pltpu.sync_copy(zeros_hbm, agg_sh)

        plsc.subcore_barrier()

        ev = [evec_v[pl.ds(16 * j, 16)] for j in range(d // 16)]
        eb = [ebias_v[pl.ds(16 * j, 16)] for j in range(d // 16)]
        ec = (ec_v.at[0], ec_v.at[1])
        att = (attr_c.at[0], attr_c.at[1])
        rows = (rows_v.at[0], rows_v.at[1])
        sems = (sem.at[0], sem.at[1])

        for b in range(2):
            pltpu.sync_copy(idx_hbm.at[base + b], ec[b])
            pltpu.sync_copy(attr_hbm.at[base + b], att[b])
            pltpu.async_copy(x_hbm.at[ec[b].at[0]], rows[b], sems[b])

        def chunk_body(i, carry):
            for b in range(2):
                k = 2 * i + b
                pltpu.make_async_copy(
                    x_hbm.at[ec[b].at[0]], rows[b], sems[b]).wait()

                def group_body(g, carry2):
                    a16 = att[b][pl.ds(g * 16, 16)]
                    i0 = g * 16
                    for i2 in range(16):
                        a = a16[i2]
                        for j in range(d // 16):
                            sl = pl.ds(16 * j, 16)
                            rows[b][i0 + i2, sl] = jnp.maximum(
                                rows[b][i0 + i2, sl] + eb[j] + a * ev[j], 0.0)
                    return carry2

                lax.fori_loop(0, C // 16, group_body, 0)
                pltpu.sync_copy(rows[b], agg_sh.at[ec[b].at[1]], add=True)

                @pl.when(k + 2 < K)
                def _():
                    pltpu.sync_copy(idx_hbm.at[base + k + 2], ec[b])
                    pltpu.sync_copy(attr_hbm.at[base + k + 2], att[b])
                    pltpu.async_copy(x_hbm.at[ec[b].at[0]], rows[b], sems[b])
            return carry

        lax.fori_loop(0, K // 2, chunk_body, 0)
        plsc.subcore_barrier()
        r0 = s * RPT
        pltpu.sync_copy(agg_sh.at[pl.ds(r0, RPT)],
                        out_hbm.at[c, pl.ds(r0, RPT)])

    return pl.kernel(
        body,
        out_type=jax.ShapeDtypeStruct((NC, N_PAD, d), jnp.float32),
        mesh=mesh,
        compiler_params=pltpu.CompilerParams(use_tc_tiling_on_sc=False),
        scratch_types=[
            pltpu.VMEM((2, 2, C), jnp.int32),
            pltpu.VMEM((2, C), jnp.float32),
            pltpu.VMEM((2, C, d), jnp.float32),
            pltpu.VMEM((d,), jnp.float32),
            pltpu.VMEM((d,), jnp.float32),
            pltpu.VMEM_SHARED((N_PAD, d), jnp.float32),
            pltpu.SemaphoreType.DMA((2,)),
        ],
    )


_edge16 = _make_edge_phase(D1)
_edge128 = _make_edge_phase(H)


def _tc_layer(din):
    def body(x_ref, agg_ref, w_ref, b_ref, o_ref):
        h = x_ref[...] + agg_ref[0] + agg_ref[1]
        y = lax.dot_general(h, w_ref[...], (((1,), (1,)), ((), ())),
                            preferred_element_type=jnp.float32)
        o_ref[...] = jnp.maximum(y + b_ref[...], 0.0)

    return pl.pallas_call(
        body,
        grid=(G,),
        in_specs=[
            pl.BlockSpec((BLK, din), lambda i: (i, 0)),
            pl.BlockSpec((NC, BLK, din), lambda i: (0, i, 0)),
            pl.BlockSpec((H, din), lambda i: (0, 0)),
            pl.BlockSpec((1, H), lambda i: (0, 0)),
        ],
        out_specs=pl.BlockSpec((BLK, H), lambda i: (i, 0)),
        out_shape=jax.ShapeDtypeStruct((N, H), jnp.float32),
    )


_layer16 = _tc_layer(D1)
_layer128 = _tc_layer(H)


def _tc_final_body(x_ref, agg_ref, w_ref, b_ref, batch_ref, wo_ref, bo_ref,
                   o_ref, sums, counts):
    i = pl.program_id(0)

    @pl.when(i == 0)
    def _():
        sums[...] = jnp.zeros_like(sums)
        counts[...] = jnp.zeros_like(counts)

    h = x_ref[...] + agg_ref[0] + agg_ref[1]
    y = lax.dot_general(h, w_ref[...], (((1,), (1,)), ((), ())),
                        preferred_element_type=jnp.float32)
    y = jnp.maximum(y + b_ref[...], 0.0)
    bt = batch_ref[0]
    ohT = (lax.broadcasted_iota(jnp.int32, (NG, BLK), 0) == bt)
    ohT = ohT.astype(jnp.float32)
    sums[...] += lax.dot_general(ohT, y, (((1,), (0,)), ((), ())),
                                 preferred_element_type=jnp.float32)
    counts[...] += jnp.sum(ohT, axis=1, keepdims=True)

    @pl.when(i == G - 1)
    def _():
        pooled = sums[...] / jnp.maximum(counts[...], 1.0)
        o_ref[...] = lax.dot_general(pooled, wo_ref[...],
                                     (((1,), (1,)), ((), ())),
                                     preferred_element_type=jnp.float32) \
            + bo_ref[...]


_final = pl.pallas_call(
    _tc_final_body,
    grid=(G,),
    in_specs=[
        pl.BlockSpec((BLK, H), lambda i: (i, 0)),
        pl.BlockSpec((NC, BLK, H), lambda i: (0, i, 0)),
        pl.BlockSpec((H, H), lambda i: (0, 0)),
        pl.BlockSpec((1, H), lambda i: (0, 0)),
        pl.BlockSpec((1, 1, BLK), lambda i: (i, 0, 0)),
        pl.BlockSpec((5, H), lambda i: (0, 0)),
        pl.BlockSpec((1, 5), lambda i: (0, 0)),
    ],
    out_specs=pl.BlockSpec((NG, 5), lambda i: (0, 0)),
    out_shape=jax.ShapeDtypeStruct((NG, 5), jnp.float32),
    scratch_shapes=[
        pltpu.VMEM((NG, H), jnp.float32),
        pltpu.VMEM((NG, 1), jnp.float32),
    ],
)


def kernel(x, edge_index, edge_attr, batch,
           W1e, b1e, W1, b1, W2e, b2e, W2, b2, W3e, b3e, W3, b3, Wo, bo):
    src = edge_index[0]
    dst = edge_index[1]
    attr = edge_attr[:, 0]
    pad = E_PAD - E
    srcs = jnp.concatenate([src, jnp.zeros((pad,), jnp.int32)])
    dsts = jnp.concatenate([dst, jnp.full((pad,), N, jnp.int32)])
    attrs = jnp.concatenate([attr, jnp.zeros((pad,), jnp.float32)])
    idxs = jnp.stack([
        srcs.reshape(E_PAD // C, C),
        dsts.reshape(E_PAD // C, C),
    ], axis=1)
    attrs = attrs.reshape(E_PAD // C, C)

    x1p = jnp.pad(x, ((0, 0), (0, D1 - 7)))
    w1p = jnp.pad(W1, ((0, 0), (0, D1 - 7)))
    ev1 = jnp.pad(W1e[:, 0], (0, D1 - 7))
    eb1 = jnp.pad(b1e, (0, D1 - 7))
    z16 = jnp.zeros((N_PAD, D1), jnp.float32)
    z128 = jnp.zeros((N_PAD, H), jnp.float32)

    agg1 = _edge16(x1p, idxs, attrs, ev1, eb1, z16)
    x2 = _layer16(x1p, agg1, w1p, b1.reshape(1, H))
    agg2 = _edge128(x2, idxs, attrs, W2e[:, 0], b2e, z128)
    x3 = _layer128(x2, agg2, W2, b2.reshape(1, H))
    agg3 = _edge128(x3, idxs, attrs, W3e[:, 0], b3e, z128)
    out = _final(x3, agg3, W3, b3.reshape(1, H),
                 batch.reshape(G, 1, BLK), Wo, bo.reshape(1, 5))
    return out


# speedup vs baseline: 7.8726x; 1.4798x over previous
import jax
import jax.numpy as jnp
from jax import lax
from jax.experimental import pallas as pl
from jax.experimental.pallas import tpu as pltpu
from jax.experimental.pallas import tpu_sc as plsc

N = 10000
E = 640000
NG = 64
H = 128
D1 = 16
NC = 2
NS = 16
NW = NC * NS
C = 128
K = 160
EW = C * K
E_PAD = EW * NW
N_PAD = 10240
RPT = N_PAD // NS
BLK = 2000
G = N // BLK


def _make_edge_phase(d):
    mesh = plsc.VectorSubcoreMesh(core_axis_name="c", subcore_axis_name="s")

    def body(x_hbm, idx_hbm, attr_hbm, evec_hbm, ebias_hbm, zeros_hbm,
             out_hbm, ec_v, attr_c, rows_v, evec_v, ebias_v, agg_sh, sem):
        c = lax.axis_index("c")
        s = lax.axis_index("s")
        base = (c * NS + s) * K
        pltpu.sync_copy(evec_hbm, evec_v)
        pltpu.sync_copy(ebias_hbm, ebias_v)

        @pl.when(s == 0)
        def _():
            pltpu.sync_copy(zeros_hbm, agg_sh)

        plsc.subcore_barrier()

        ev = [evec_v[pl.ds(16 * j, 16)] for j in range(d // 16)]
        eb = [ebias_v[pl.ds(16 * j, 16)] for j in range(d // 16)]
        ec = (ec_v.at[0], ec_v.at[1])
        att = (attr_c.at[0], attr_c.at[1])
        rows = (rows_v.at[0], rows_v.at[1])
        sems = (sem.at[0], sem.at[1])

        for b in range(2):
            pltpu.sync_copy(idx_hbm.at[base + b], ec[b])
            pltpu.sync_copy(attr_hbm.at[base + b], att[b])
            pltpu.async_copy(x_hbm.at[ec[b].at[0]], rows[b], sems[b])

        def chunk_body(i, carry):
            for b in range(2):
                k = 2 * i + b
                pltpu.make_async_copy(
                    x_hbm.at[ec[b].at[0]], rows[b], sems[b]).wait()

                def group_body(g, carry2):
                    a16 = att[b][pl.ds(g * 16, 16)]
                    i0 = g * 16
                    for i2 in range(16):
                        a = a16[i2]
                        for j in range(d // 16):
                            sl = pl.ds(16 * j, 16)
                            rows[b][i0 + i2, sl] = jnp.maximum(
                                rows[b][i0 + i2, sl] + eb[j] + a * ev[j], 0.0)
                    return carry2

                lax.fori_loop(0, C // 16, group_body, 0)
                pltpu.sync_copy(rows[b], agg_sh.at[ec[b].at[1]], add=True)

                @pl.when(k + 2 < K)
                def _():
                    pltpu.sync_copy(idx_hbm.at[base + k + 2], ec[b])
                    pltpu.sync_copy(attr_hbm.at[base + k + 2], att[b])
                    pltpu.async_copy(x_hbm.at[ec[b].at[0]], rows[b], sems[b])
            return carry

        lax.fori_loop(0, K // 2, chunk_body, 0)
        plsc.subcore_barrier()
        r0 = s * RPT
        pltpu.sync_copy(agg_sh.at[pl.ds(r0, RPT)],
                        out_hbm.at[c, pl.ds(r0, RPT)])

    return pl.kernel(
        body,
        out_type=jax.ShapeDtypeStruct((NC, N_PAD, d), jnp.float32),
        mesh=mesh,
        compiler_params=pltpu.CompilerParams(use_tc_tiling_on_sc=False),
        scratch_types=[
            pltpu.VMEM((2, 2, C), jnp.int32),
            pltpu.VMEM((2, C), jnp.float32),
            pltpu.VMEM((2, C, d), jnp.float32),
            pltpu.VMEM((d,), jnp.float32),
            pltpu.VMEM((d,), jnp.float32),
            pltpu.VMEM_SHARED((N_PAD, d), jnp.float32),
            pltpu.SemaphoreType.DMA((2,)),
        ],
    )


_edge16 = _make_edge_phase(D1)
_edge128 = _make_edge_phase(H)


def _tc_layer(din):
    def body(x_ref, agg_ref, w_ref, b_ref, o_ref):
        h = x_ref[...] + agg_ref[0] + agg_ref[1]
        y = lax.dot_general(h, w_ref[...], (((1,), (1,)), ((), ())),
                            preferred_element_type=jnp.float32)
        o_ref[...] = jnp.maximum(y + b_ref[...], 0.0)

    return pl.pallas_call(
        body,
        grid=(G,),
        in_specs=[
            pl.BlockSpec((BLK, din), lambda i: (i, 0)),
            pl.BlockSpec((NC, BLK, din), lambda i: (0, i, 0)),
            pl.BlockSpec((H, din), lambda i: (0, 0)),
            pl.BlockSpec((1, H), lambda i: (0, 0)),
        ],
        out_specs=pl.BlockSpec((BLK, H), lambda i: (i, 0)),
        out_shape=jax.ShapeDtypeStruct((N, H), jnp.float32),
    )


_layer16 = _tc_layer(D1)
_layer128 = _tc_layer(H)


def _tc_final_body(x_ref, agg_ref, w_ref, b_ref, batch_ref, wo_ref, bo_ref,
                   o_ref, sums, counts):
    i = pl.program_id(0)

    @pl.when(i == 0)
    def _():
        sums[...] = jnp.zeros_like(sums)
        counts[...] = jnp.zeros_like(counts)

    h = x_ref[...] + agg_ref[0] + agg_ref[1]
    y = lax.dot_general(h, w_ref[...], (((1,), (1,)), ((), ())),
                        preferred_element_type=jnp.float32)
    y = jnp.maximum(y + b_ref[...], 0.0)
    bt = batch_ref[0]
    ohT = (lax.broadcasted_iota(jnp.int32, (NG, BLK), 0) == bt)
    ohT = ohT.astype(jnp.float32)
    sums[...] += lax.dot_general(ohT, y, (((1,), (0,)), ((), ())),
                                 preferred_element_type=jnp.float32)
    counts[...] += jnp.sum(ohT, axis=1, keepdims=True)

    @pl.when(i == G - 1)
    def _():
        pooled = sums[...] / jnp.maximum(counts[...], 1.0)
        o_ref[...] = lax.dot_general(pooled, wo_ref[...],
                                     (((1,), (1,)), ((), ())),
                                     preferred_element_type=jnp.float32) \
            + bo_ref[...]


_final = pl.pallas_call(
    _tc_final_body,
    grid=(G,),
    in_specs=[
        pl.BlockSpec((BLK, H), lambda i: (i, 0)),
        pl.BlockSpec((NC, BLK, H), lambda i: (0, i, 0)),
        pl.BlockSpec((H, H), lambda i: (0, 0)),
        pl.BlockSpec((1, H), lambda i: (0, 0)),
        pl.BlockSpec((1, 1, BLK), lambda i: (i, 0, 0)),
        pl.BlockSpec((5, H), lambda i: (0, 0)),
        pl.BlockSpec((1, 5), lambda i: (0, 0)),
    ],
    out_specs=pl.BlockSpec((NG, 5), lambda i: (0, 0)),
    out_shape=jax.ShapeDtypeStruct((NG, 5), jnp.float32),
    scratch_shapes=[
        pltpu.VMEM((NG, H), jnp.float32),
        pltpu.VMEM((NG, 1), jnp.float32),
    ],
)


def kernel(x, edge_index, edge_attr, batch,
           W1e, b1e, W1, b1, W2e, b2e, W2, b2, W3e, b3e, W3, b3, Wo, bo):
    src = edge_index[0]
    dst = edge_index[1]
    attr = edge_attr[:, 0]
    pad = E_PAD - E
    fill = jnp.arange(pad, dtype=jnp.int32)
    srcs = jnp.concatenate([src, fill % N])
    dsts = jnp.concatenate([dst, N + fill % (N_PAD - N)])
    attrs = jnp.concatenate([attr, jnp.zeros((pad,), jnp.float32)])
    idxs = jnp.stack([
        srcs.reshape(E_PAD // C, C),
        dsts.reshape(E_PAD // C, C),
    ], axis=1)
    attrs = attrs.reshape(E_PAD // C, C)

    x1p = jnp.pad(x, ((0, 0), (0, D1 - 7)))
    w1p = jnp.pad(W1, ((0, 0), (0, D1 - 7)))
    ev1 = jnp.pad(W1e[:, 0], (0, D1 - 7))
    eb1 = jnp.pad(b1e, (0, D1 - 7))
    z16 = jnp.zeros((N_PAD, D1), jnp.float32)
    z128 = jnp.zeros((N_PAD, H), jnp.float32)

    agg1 = _edge16(x1p, idxs, attrs, ev1, eb1, z16)
    x2 = _layer16(x1p, agg1, w1p, b1.reshape(1, H))
    agg2 = _edge128(x2, idxs, attrs, W2e[:, 0], b2e, z128)
    x3 = _layer128(x2, agg2, W2, b2.reshape(1, H))
    agg3 = _edge128(x3, idxs, attrs, W3e[:, 0], b3e, z128)
    out = _final(x3, agg3, W3, b3.reshape(1, H),
                 batch.reshape(G, 1, BLK), Wo, bo.reshape(1, 5))
    return out


# speedup vs baseline: 7.9350x; 1.0079x over previous
import jax
import jax.numpy as jnp
from jax import lax
from jax.experimental import pallas as pl
from jax.experimental.pallas import tpu as pltpu
from jax.experimental.pallas import tpu_sc as plsc

N = 10000
E = 640000
NG = 64
H = 128
D1 = 16
NC = 2
NS = 16
NW = NC * NS
C = 96
K = 216
EW = C * K
E_PAD = EW * NW
N_PAD = 10240
RPT = N_PAD // NS
BLK = 2000
G = N // BLK


def _make_edge_phase(d, ck, kk):
    mesh = plsc.VectorSubcoreMesh(core_axis_name="c", subcore_axis_name="s")

    def body(x_hbm, idx_hbm, attr_hbm, evec_hbm, ebias_hbm, zeros_hbm,
             out_hbm, ec_v, attr_c, rows_v, evec_v, ebias_v, agg_sh,
             gsem, ssem):
        c = lax.axis_index("c")
        s = lax.axis_index("s")
        base = (c * NS + s) * kk
        pltpu.sync_copy(evec_hbm, evec_v)
        pltpu.sync_copy(ebias_hbm, ebias_v)

        @pl.when(s == 0)
        def _():
            pltpu.sync_copy(zeros_hbm, agg_sh)

        plsc.subcore_barrier()

        ev = [evec_v[pl.ds(16 * j, 16)] for j in range(d // 16)]
        eb = [ebias_v[pl.ds(16 * j, 16)] for j in range(d // 16)]
        ec = tuple(ec_v.at[b] for b in range(3))
        att = tuple(attr_c.at[b] for b in range(3))
        rows = tuple(rows_v.at[b] for b in range(3))
        gs = tuple(gsem.at[b] for b in range(3))
        ss = tuple(ssem.at[b] for b in range(3))

        for b in range(2):
            pltpu.sync_copy(idx_hbm.at[base + b], ec[b])
            pltpu.sync_copy(attr_hbm.at[base + b], att[b])
            pltpu.async_copy(x_hbm.at[ec[b].at[0]], rows[b], gs[b])

        def chunk_body(i, carry):
            for b in range(3):
                k = 3 * i + b
                b2 = (b + 2) % 3
                pltpu.make_async_copy(
                    x_hbm.at[ec[b].at[0]], rows[b], gs[b]).wait()

                def group_body(g, carry2):
                    a16 = att[b][pl.ds(g * 16, 16)]
                    i0 = g * 16
                    for i2 in range(16):
                        a = a16[i2]
                        for j in range(d // 16):
                            sl = pl.ds(16 * j, 16)
                            rows[b][i0 + i2, sl] = jnp.maximum(
                                rows[b][i0 + i2, sl] + eb[j] + a * ev[j], 0.0)
                    return carry2

                lax.fori_loop(0, ck // 16, group_body, 0)
                pltpu.async_copy(rows[b], agg_sh.at[ec[b].at[1]], ss[b],
                                 add=True)

                @pl.when(k >= 1)
                def _():
                    pltpu.make_async_copy(
                        rows[b2], agg_sh.at[ec[b2].at[1]], ss[b2]).wait()

                @pl.when(k + 2 < kk)
                def _():
                    pltpu.sync_copy(idx_hbm.at[base + k + 2], ec[b2])
                    pltpu.sync_copy(attr_hbm.at[base + k + 2], att[b2])
                    pltpu.async_copy(x_hbm.at[ec[b2].at[0]], rows[b2],
                                     gs[b2])
            return carry

        lax.fori_loop(0, kk // 3, chunk_body, 0)
        bl = (kk - 1) % 3
        pltpu.make_async_copy(rows[bl], agg_sh.at[ec[bl].at[1]],
                              ss[bl]).wait()
        plsc.subcore_barrier()
        r0 = s * RPT
        pltpu.sync_copy(agg_sh.at[pl.ds(r0, RPT)],
                        out_hbm.at[c, pl.ds(r0, RPT)])

    return pl.kernel(
        body,
        out_type=jax.ShapeDtypeStruct((NC, N_PAD, d), jnp.float32),
        mesh=mesh,
        compiler_params=pltpu.CompilerParams(use_tc_tiling_on_sc=False),
        scratch_types=[
            pltpu.VMEM((3, 2, ck), jnp.int32),
            pltpu.VMEM((3, ck), jnp.float32),
            pltpu.VMEM((3, ck, d), jnp.float32),
            pltpu.VMEM((d,), jnp.float32),
            pltpu.VMEM((d,), jnp.float32),
            pltpu.VMEM_SHARED((N_PAD, d), jnp.float32),
            pltpu.SemaphoreType.DMA((3,)),
            pltpu.SemaphoreType.DMA((3,)),
        ],
    )


C1 = 128
K1 = 162
_edge16 = _make_edge_phase(D1, C1, K1)
_edge128 = _make_edge_phase(H, C, K)


def _tc_layer(din):
    def body(x_ref, agg_ref, w_ref, b_ref, o_ref):
        h = x_ref[...] + agg_ref[0] + agg_ref[1]
        y = lax.dot_general(h, w_ref[...], (((1,), (1,)), ((), ())),
                            preferred_element_type=jnp.float32)
        o_ref[...] = jnp.maximum(y + b_ref[...], 0.0)

    return pl.pallas_call(
        body,
        grid=(G,),
        in_specs=[
            pl.BlockSpec((BLK, din), lambda i: (i, 0)),
            pl.BlockSpec((NC, BLK, din), lambda i: (0, i, 0)),
            pl.BlockSpec((H, din), lambda i: (0, 0)),
            pl.BlockSpec((1, H), lambda i: (0, 0)),
        ],
        out_specs=pl.BlockSpec((BLK, H), lambda i: (i, 0)),
        out_shape=jax.ShapeDtypeStruct((N, H), jnp.float32),
    )


_layer16 = _tc_layer(D1)
_layer128 = _tc_layer(H)


def _tc_final_body(x_ref, agg_ref, w_ref, b_ref, batch_ref, wo_ref, bo_ref,
                   o_ref, sums, counts):
    i = pl.program_id(0)

    @pl.when(i == 0)
    def _():
        sums[...] = jnp.zeros_like(sums)
        counts[...] = jnp.zeros_like(counts)

    h = x_ref[...] + agg_ref[0] + agg_ref[1]
    y = lax.dot_general(h, w_ref[...], (((1,), (1,)), ((), ())),
                        preferred_element_type=jnp.float32)
    y = jnp.maximum(y + b_ref[...], 0.0)
    bt = batch_ref[0]
    ohT = (lax.broadcasted_iota(jnp.int32, (NG, BLK), 0) == bt)
    ohT = ohT.astype(jnp.float32)
    sums[...] += lax.dot_general(ohT, y, (((1,), (0,)), ((), ())),
                                 preferred_element_type=jnp.float32)
    counts[...] += jnp.sum(ohT, axis=1, keepdims=True)

    @pl.when(i == G - 1)
    def _():
        pooled = sums[...] / jnp.maximum(counts[...], 1.0)
        o_ref[...] = lax.dot_general(pooled, wo_ref[...],
                                     (((1,), (1,)), ((), ())),
                                     preferred_element_type=jnp.float32) \
            + bo_ref[...]


_final = pl.pallas_call(
    _tc_final_body,
    grid=(G,),
    in_specs=[
        pl.BlockSpec((BLK, H), lambda i: (i, 0)),
        pl.BlockSpec((NC, BLK, H), lambda i: (0, i, 0)),
        pl.BlockSpec((H, H), lambda i: (0, 0)),
        pl.BlockSpec((1, H), lambda i: (0, 0)),
        pl.BlockSpec((1, 1, BLK), lambda i: (i, 0, 0)),
        pl.BlockSpec((5, H), lambda i: (0, 0)),
        pl.BlockSpec((1, 5), lambda i: (0, 0)),
    ],
    out_specs=pl.BlockSpec((NG, 5), lambda i: (0, 0)),
    out_shape=jax.ShapeDtypeStruct((NG, 5), jnp.float32),
    scratch_shapes=[
        pltpu.VMEM((NG, H), jnp.float32),
        pltpu.VMEM((NG, 1), jnp.float32),
    ],
)


def kernel(x, edge_index, edge_attr, batch,
           W1e, b1e, W1, b1, W2e, b2e, W2, b2, W3e, b3e, W3, b3, Wo, bo):
    src = edge_index[0]
    dst = edge_index[1]
    attr = edge_attr[:, 0]
    pad = E_PAD - E
    fill = jnp.arange(pad, dtype=jnp.int32)
    srcs = jnp.concatenate([src, fill % N])
    dsts = jnp.concatenate([dst, N + fill % (N_PAD - N)])
    attrs = jnp.concatenate([attr, jnp.zeros((pad,), jnp.float32)])
    idxs = jnp.stack([
        srcs.reshape(E_PAD // C, C),
        dsts.reshape(E_PAD // C, C),
    ], axis=1)
    idxs1 = jnp.stack([
        srcs.reshape(E_PAD // C1, C1),
        dsts.reshape(E_PAD // C1, C1),
    ], axis=1)
    attrs1 = attrs.reshape(E_PAD // C1, C1)
    attrs = attrs.reshape(E_PAD // C, C)

    x1p = jnp.pad(x, ((0, 0), (0, D1 - 7)))
    w1p = jnp.pad(W1, ((0, 0), (0, D1 - 7)))
    ev1 = jnp.pad(W1e[:, 0], (0, D1 - 7))
    eb1 = jnp.pad(b1e, (0, D1 - 7))
    z16 = jnp.zeros((N_PAD, D1), jnp.float32)
    z128 = jnp.zeros((N_PAD, H), jnp.float32)

    agg1 = _edge16(x1p, idxs1, attrs1, ev1, eb1, z16)
    x2 = _layer16(x1p, agg1, w1p, b1.reshape(1, H))
    agg2 = _edge128(x2, idxs, attrs, W2e[:, 0], b2e, z128)
    x3 = _layer128(x2, agg2, W2, b2.reshape(1, H))
    agg3 = _edge128(x3, idxs, attrs, W3e[:, 0], b3e, z128)
    out = _final(x3, agg3, W3, b3.reshape(1, H),
                 batch.reshape(G, 1, BLK), Wo, bo.reshape(1, 5))
    return out


# speedup vs baseline: 13.0104x; 1.6396x over previous
import jax
import jax.numpy as jnp
from jax import lax
from jax.experimental import pallas as pl
from jax.experimental.pallas import tpu as pltpu
from jax.experimental.pallas import tpu_sc as plsc

N = 10000
E = 640000
NG = 64
H = 128
D1 = 16
NC = 2
NS = 16
NW = NC * NS
C = 96
K = 216
EW = C * K
E_PAD = EW * NW
N_PAD = 10240
RPT = N_PAD // NS
BLK = 2000
G = N // BLK


def _make_edge_phase(d, ck, kk):
    mesh = plsc.VectorSubcoreMesh(core_axis_name="c", subcore_axis_name="s")

    def body(x_hbm, idx_hbm, attr_hbm, evec_hbm, zeros_hbm,
             out_hbm, ec_v, attr_c, rows_v, evec_v, agg_sh,
             gsem, ssem):
        c = lax.axis_index("c")
        s = lax.axis_index("s")
        base = (c * NS + s) * kk
        pltpu.sync_copy(evec_hbm, evec_v)

        @pl.when(s == 0)
        def _():
            pltpu.sync_copy(zeros_hbm, agg_sh)

        plsc.subcore_barrier()

        ev = [evec_v[pl.ds(16 * j, 16)] for j in range(d // 16)]
        ec = tuple(ec_v.at[b] for b in range(3))
        att = tuple(attr_c.at[b] for b in range(3))
        rows = tuple(rows_v.at[b] for b in range(3))
        gs = tuple(gsem.at[b] for b in range(3))
        ss = tuple(ssem.at[b] for b in range(3))

        for b in range(2):
            pltpu.sync_copy(idx_hbm.at[base + b], ec[b])
            pltpu.sync_copy(attr_hbm.at[base + b], att[b])
            pltpu.async_copy(x_hbm.at[ec[b].at[0]], rows[b], gs[b])

        def chunk_body(i, carry):
            for b in range(3):
                k = 3 * i + b
                b2 = (b + 2) % 3
                pltpu.make_async_copy(
                    x_hbm.at[ec[b].at[0]], rows[b], gs[b]).wait()

                @plsc.parallel_loop(0, ck // 16, unroll=2)
                def group_body(g):
                    a16 = att[b][pl.ds(g * 16, 16)]
                    i0 = g * 16
                    for i2 in range(16):
                        a = a16[i2]
                        for j in range(d // 16):
                            sl = pl.ds(16 * j, 16)
                            rows[b][i0 + i2, sl] = jnp.maximum(
                                rows[b][i0 + i2, sl] + a * ev[j], 0.0)
                pltpu.async_copy(rows[b], agg_sh.at[ec[b].at[1]], ss[b],
                                 add=True)

                @pl.when(k >= 1)
                def _():
                    pltpu.make_async_copy(
                        rows[b2], agg_sh.at[ec[b2].at[1]], ss[b2]).wait()

                @pl.when(k + 2 < kk)
                def _():
                    pltpu.sync_copy(idx_hbm.at[base + k + 2], ec[b2])
                    pltpu.sync_copy(attr_hbm.at[base + k + 2], att[b2])
                    pltpu.async_copy(x_hbm.at[ec[b2].at[0]], rows[b2],
                                     gs[b2])
            return carry

        lax.fori_loop(0, kk // 3, chunk_body, 0)
        bl = (kk - 1) % 3
        pltpu.make_async_copy(rows[bl], agg_sh.at[ec[bl].at[1]],
                              ss[bl]).wait()
        plsc.subcore_barrier()
        r0 = s * RPT
        pltpu.sync_copy(agg_sh.at[pl.ds(r0, RPT)],
                        out_hbm.at[c, pl.ds(r0, RPT)])

    return pl.kernel(
        body,
        out_type=jax.ShapeDtypeStruct((NC, N_PAD, d), jnp.float32),
        mesh=mesh,
        compiler_params=pltpu.CompilerParams(use_tc_tiling_on_sc=False),
        scratch_types=[
            pltpu.VMEM((3, 2, ck), jnp.int32),
            pltpu.VMEM((3, ck), jnp.float32),
            pltpu.VMEM((3, ck, d), jnp.float32),
            pltpu.VMEM((d,), jnp.float32),
            pltpu.VMEM_SHARED((N_PAD, d), jnp.float32),
            pltpu.SemaphoreType.DMA((3,)),
            pltpu.SemaphoreType.DMA((3,)),
        ],
    )


C1 = 128
K1 = 162
_edge16 = _make_edge_phase(D1, C1, K1)
_edge128 = _make_edge_phase(H, C, K)


def _tc_layer(din):
    def body(x_ref, agg_ref, w_ref, b_ref, ebn_ref, o_ref, ot_ref):
        h = x_ref[...] + agg_ref[0] + agg_ref[1]
        y = lax.dot_general(h, w_ref[...], (((1,), (1,)), ((), ())),
                            preferred_element_type=jnp.float32)
        y = jnp.maximum(y + b_ref[...], 0.0)
        o_ref[...] = y
        ot_ref[...] = y + ebn_ref[...]

    return pl.pallas_call(
        body,
        grid=(G,),
        in_specs=[
            pl.BlockSpec((BLK, din), lambda i: (i, 0)),
            pl.BlockSpec((NC, BLK, din), lambda i: (0, i, 0)),
            pl.BlockSpec((H, din), lambda i: (0, 0)),
            pl.BlockSpec((1, H), lambda i: (0, 0)),
            pl.BlockSpec((1, H), lambda i: (0, 0)),
        ],
        out_specs=[pl.BlockSpec((BLK, H), lambda i: (i, 0)),
                   pl.BlockSpec((BLK, H), lambda i: (i, 0))],
        out_shape=[jax.ShapeDtypeStruct((N, H), jnp.float32),
                   jax.ShapeDtypeStruct((N, H), jnp.float32)],
    )


_layer16 = _tc_layer(D1)
_layer128 = _tc_layer(H)


def _tc_final_body(x_ref, agg_ref, w_ref, b_ref, batch_ref, wo_ref, bo_ref,
                   o_ref, sums, counts):
    i = pl.program_id(0)

    @pl.when(i == 0)
    def _():
        sums[...] = jnp.zeros_like(sums)
        counts[...] = jnp.zeros_like(counts)

    h = x_ref[...] + agg_ref[0] + agg_ref[1]
    y = lax.dot_general(h, w_ref[...], (((1,), (1,)), ((), ())),
                        preferred_element_type=jnp.float32)
    y = jnp.maximum(y + b_ref[...], 0.0)
    bt = batch_ref[0]
    ohT = (lax.broadcasted_iota(jnp.int32, (NG, BLK), 0) == bt)
    ohT = ohT.astype(jnp.float32)
    sums[...] += lax.dot_general(ohT, y, (((1,), (0,)), ((), ())),
                                 preferred_element_type=jnp.float32)
    counts[...] += jnp.sum(ohT, axis=1, keepdims=True)

    @pl.when(i == G - 1)
    def _():
        pooled = sums[...] / jnp.maximum(counts[...], 1.0)
        o_ref[...] = lax.dot_general(pooled, wo_ref[...],
                                     (((1,), (1,)), ((), ())),
                                     preferred_element_type=jnp.float32) \
            + bo_ref[...]


_final = pl.pallas_call(
    _tc_final_body,
    grid=(G,),
    in_specs=[
        pl.BlockSpec((BLK, H), lambda i: (i, 0)),
        pl.BlockSpec((NC, BLK, H), lambda i: (0, i, 0)),
        pl.BlockSpec((H, H), lambda i: (0, 0)),
        pl.BlockSpec((1, H), lambda i: (0, 0)),
        pl.BlockSpec((1, 1, BLK), lambda i: (i, 0, 0)),
        pl.BlockSpec((5, H), lambda i: (0, 0)),
        pl.BlockSpec((1, 5), lambda i: (0, 0)),
    ],
    out_specs=pl.BlockSpec((NG, 5), lambda i: (0, 0)),
    out_shape=jax.ShapeDtypeStruct((NG, 5), jnp.float32),
    scratch_shapes=[
        pltpu.VMEM((NG, H), jnp.float32),
        pltpu.VMEM((NG, 1), jnp.float32),
    ],
)


def kernel(x, edge_index, edge_attr, batch,
           W1e, b1e, W1, b1, W2e, b2e, W2, b2, W3e, b3e, W3, b3, Wo, bo):
    src = edge_index[0]
    dst = edge_index[1]
    attr = edge_attr[:, 0]
    pad = E_PAD - E
    fill = jnp.arange(pad, dtype=jnp.int32)
    srcs = jnp.concatenate([src, fill % N])
    dsts = jnp.concatenate([dst, N + fill % (N_PAD - N)])
    attrs = jnp.concatenate([attr, jnp.zeros((pad,), jnp.float32)])
    idxs = jnp.stack([
        srcs.reshape(E_PAD // C, C),
        dsts.reshape(E_PAD // C, C),
    ], axis=1)
    idxs1 = jnp.stack([
        srcs.reshape(E_PAD // C1, C1),
        dsts.reshape(E_PAD // C1, C1),
    ], axis=1)
    attrs1 = attrs.reshape(E_PAD // C1, C1)
    attrs = attrs.reshape(E_PAD // C, C)

    x1p = jnp.pad(x, ((0, 0), (0, D1 - 7)))
    w1p = jnp.pad(W1, ((0, 0), (0, D1 - 7)))
    ev1 = jnp.pad(W1e[:, 0], (0, D1 - 7))
    x1t = x1p + jnp.pad(b1e, (0, D1 - 7))[None, :]
    z16 = jnp.zeros((N_PAD, D1), jnp.float32)
    z128 = jnp.zeros((N_PAD, H), jnp.float32)

    agg1 = _edge16(x1t, idxs1, attrs1, ev1, z16)
    x2, x2t = _layer16(x1p, agg1, w1p, b1.reshape(1, H), b2e.reshape(1, H))
    agg2 = _edge128(x2t, idxs, attrs, W2e[:, 0], z128)
    x3, x3t = _layer128(x2, agg2, W2, b2.reshape(1, H), b3e.reshape(1, H))
    agg3 = _edge128(x3t, idxs, attrs, W3e[:, 0], z128)
    out = _final(x3, agg3, W3, b3.reshape(1, H),
                 batch.reshape(G, 1, BLK), Wo, bo.reshape(1, 5))
    return out


# speedup vs baseline: 15.6043x; 1.1994x over previous
import jax
import jax.numpy as jnp
from jax import lax
from jax.experimental import pallas as pl
from jax.experimental.pallas import tpu as pltpu
from jax.experimental.pallas import tpu_sc as plsc

N = 10000
E = 640000
NG = 64
H = 128
D1 = 16
NC = 2
NS = 16
NW = NC * NS
C = 96
K = 216
EW = C * K
E_PAD = EW * NW
N_PAD = 10240
RPT = N_PAD // NS
BLK = 2000
G = N // BLK


def _make_edge_phase(d, ck, kk, sub):
    mesh = plsc.VectorSubcoreMesh(core_axis_name="c", subcore_axis_name="s")
    sc = sub * ck

    def body(x_hbm, idx_hbm, attr_hbm, evec_hbm, zeros_hbm,
             out_hbm, ec_v, attr_c, rows_v, evec_v, agg_sh,
             gsem, ssem):
        c = lax.axis_index("c")
        s = lax.axis_index("s")
        base = (c * NS + s) * kk
        pltpu.sync_copy(evec_hbm, evec_v)

        @pl.when(s == 0)
        def _():
            pltpu.sync_copy(zeros_hbm, agg_sh)

        plsc.subcore_barrier()

        ev = [evec_v[pl.ds(16 * j, 16)] for j in range(d // 16)]
        ec = tuple(ec_v.at[b] for b in range(3))
        att = tuple(attr_c.at[b] for b in range(3))
        rows = tuple(rows_v.at[b] for b in range(3))
        gs = tuple(gsem.at[b] for b in range(3))
        ss = tuple(ssem.at[b] for b in range(3))

        def launch(b, k):
            pltpu.sync_copy(idx_hbm.at[base + k], ec[b])
            pltpu.sync_copy(attr_hbm.at[base + k], att[b])
            for m in range(sub):
                pltpu.async_copy(x_hbm.at[ec[b].at[0, m]],
                                 rows[b].at[pl.ds(m * ck, ck)], gs[b])

        def drain_gather(b):
            for m in range(sub):
                pltpu.make_async_copy(x_hbm.at[ec[b].at[0, m]],
                                      rows[b].at[pl.ds(m * ck, ck)],
                                      gs[b]).wait()

        def start_scatter(b):
            for m in range(sub):
                pltpu.async_copy(rows[b].at[pl.ds(m * ck, ck)],
                                 agg_sh.at[ec[b].at[1, m]], ss[b], add=True)

        def drain_scatter(b):
            for m in range(sub):
                pltpu.make_async_copy(rows[b].at[pl.ds(m * ck, ck)],
                                      agg_sh.at[ec[b].at[1, m]],
                                      ss[b]).wait()

        for b in range(2):
            launch(b, b)

        def slot_body(i, carry):
            for b in range(3):
                k = 3 * i + b
                b2 = (b + 2) % 3
                drain_gather(b)

                @plsc.parallel_loop(0, sc // 16, unroll=4)
                def group_body(g):
                    a16 = att[b][pl.ds(g * 16, 16)]
                    i0 = g * 16
                    for i2 in range(16):
                        a = a16[i2]
                        for j in range(d // 16):
                            sl = pl.ds(16 * j, 16)
                            rows[b][i0 + i2, sl] = jnp.maximum(
                                rows[b][i0 + i2, sl] + a * ev[j], 0.0)
                start_scatter(b)

                @pl.when(k >= 1)
                def _():
                    drain_scatter(b2)

                @pl.when(k + 2 < kk)
                def _():
                    launch(b2, k + 2)
            return carry

        lax.fori_loop(0, kk // 3, slot_body, 0)
        drain_scatter((kk - 1) % 3)
        plsc.subcore_barrier()
        r0 = s * RPT
        pltpu.sync_copy(agg_sh.at[pl.ds(r0, RPT)],
                        out_hbm.at[c, pl.ds(r0, RPT)])

    return pl.kernel(
        body,
        out_type=jax.ShapeDtypeStruct((NC, N_PAD, d), jnp.float32),
        mesh=mesh,
        compiler_params=pltpu.CompilerParams(use_tc_tiling_on_sc=False),
        scratch_types=[
            pltpu.VMEM((3, 2, sub, ck), jnp.int32),
            pltpu.VMEM((3, sc), jnp.float32),
            pltpu.VMEM((3, sc, d), jnp.float32),
            pltpu.VMEM((d,), jnp.float32),
            pltpu.VMEM_SHARED((N_PAD, d), jnp.float32),
            pltpu.SemaphoreType.DMA((3,)),
            pltpu.SemaphoreType.DMA((3,)),
        ],
    )


C1 = 128
S1 = 6
K1 = 27
_edge16 = _make_edge_phase(D1, C1, K1, S1)
_edge128 = _make_edge_phase(H, C, K, 1)


def _tc_layer(din):
    def body(x_ref, agg_ref, w_ref, b_ref, ebn_ref, o_ref, ot_ref):
        h = x_ref[...] + agg_ref[0] + agg_ref[1]
        y = lax.dot_general(h, w_ref[...], (((1,), (1,)), ((), ())),
                            preferred_element_type=jnp.float32)
        y = jnp.maximum(y + b_ref[...], 0.0)
        o_ref[...] = y
        ot_ref[...] = y + ebn_ref[...]

    return pl.pallas_call(
        body,
        grid=(G,),
        in_specs=[
            pl.BlockSpec((BLK, din), lambda i: (i, 0)),
            pl.BlockSpec((NC, BLK, din), lambda i: (0, i, 0)),
            pl.BlockSpec((H, din), lambda i: (0, 0)),
            pl.BlockSpec((1, H), lambda i: (0, 0)),
            pl.BlockSpec((1, H), lambda i: (0, 0)),
        ],
        out_specs=[pl.BlockSpec((BLK, H), lambda i: (i, 0)),
                   pl.BlockSpec((BLK, H), lambda i: (i, 0))],
        out_shape=[jax.ShapeDtypeStruct((N, H), jnp.float32),
                   jax.ShapeDtypeStruct((N, H), jnp.float32)],
    )


_layer16 = _tc_layer(D1)
_layer128 = _tc_layer(H)


def _tc_final_body(x_ref, agg_ref, w_ref, b_ref, batch_ref, wo_ref, bo_ref,
                   o_ref, sums, counts):
    i = pl.program_id(0)

    @pl.when(i == 0)
    def _():
        sums[...] = jnp.zeros_like(sums)
        counts[...] = jnp.zeros_like(counts)

    h = x_ref[...] + agg_ref[0] + agg_ref[1]
    y = lax.dot_general(h, w_ref[...], (((1,), (1,)), ((), ())),
                        preferred_element_type=jnp.float32)
    y = jnp.maximum(y + b_ref[...], 0.0)
    bt = batch_ref[0]
    ohT = (lax.broadcasted_iota(jnp.int32, (NG, BLK), 0) == bt)
    ohT = ohT.astype(jnp.float32)
    sums[...] += lax.dot_general(ohT, y, (((1,), (0,)), ((), ())),
                                 preferred_element_type=jnp.float32)
    counts[...] += jnp.sum(ohT, axis=1, keepdims=True)

    @pl.when(i == G - 1)
    def _():
        pooled = sums[...] / jnp.maximum(counts[...], 1.0)
        o_ref[...] = lax.dot_general(pooled, wo_ref[...],
                                     (((1,), (1,)), ((), ())),
                                     preferred_element_type=jnp.float32) \
            + bo_ref[...]


_final = pl.pallas_call(
    _tc_final_body,
    grid=(G,),
    in_specs=[
        pl.BlockSpec((BLK, H), lambda i: (i, 0)),
        pl.BlockSpec((NC, BLK, H), lambda i: (0, i, 0)),
        pl.BlockSpec((H, H), lambda i: (0, 0)),
        pl.BlockSpec((1, H), lambda i: (0, 0)),
        pl.BlockSpec((1, 1, BLK), lambda i: (i, 0, 0)),
        pl.BlockSpec((5, H), lambda i: (0, 0)),
        pl.BlockSpec((1, 5), lambda i: (0, 0)),
    ],
    out_specs=pl.BlockSpec((NG, 5), lambda i: (0, 0)),
    out_shape=jax.ShapeDtypeStruct((NG, 5), jnp.float32),
    scratch_shapes=[
        pltpu.VMEM((NG, H), jnp.float32),
        pltpu.VMEM((NG, 1), jnp.float32),
    ],
)


def kernel(x, edge_index, edge_attr, batch,
           W1e, b1e, W1, b1, W2e, b2e, W2, b2, W3e, b3e, W3, b3, Wo, bo):
    src = edge_index[0]
    dst = edge_index[1]
    attr = edge_attr[:, 0]
    pad = E_PAD - E
    fill = jnp.arange(pad, dtype=jnp.int32)
    srcs = jnp.concatenate([src, fill % N])
    dsts = jnp.concatenate([dst, N + fill % (N_PAD - N)])
    attrs = jnp.concatenate([attr, jnp.zeros((pad,), jnp.float32)])
    idxs = jnp.stack([
        srcs.reshape(E_PAD // C, 1, C),
        dsts.reshape(E_PAD // C, 1, C),
    ], axis=1)
    sc1 = S1 * C1
    idxs1 = jnp.stack([
        srcs.reshape(E_PAD // sc1, S1, C1),
        dsts.reshape(E_PAD // sc1, S1, C1),
    ], axis=1)
    attrs1 = attrs.reshape(E_PAD // sc1, sc1)
    attrs = attrs.reshape(E_PAD // C, C)

    x1p = jnp.pad(x, ((0, 0), (0, D1 - 7)))
    w1p = jnp.pad(W1, ((0, 0), (0, D1 - 7)))
    ev1 = jnp.pad(W1e[:, 0], (0, D1 - 7))
    x1t = x1p + jnp.pad(b1e, (0, D1 - 7))[None, :]
    z16 = jnp.zeros((N_PAD, D1), jnp.float32)
    z128 = jnp.zeros((N_PAD, H), jnp.float32)

    agg1 = _edge16(x1t, idxs1, attrs1, ev1, z16)
    x2, x2t = _layer16(x1p, agg1, w1p, b1.reshape(1, H), b2e.reshape(1, H))
    agg2 = _edge128(x2t, idxs, attrs, W2e[:, 0], z128)
    x3, x3t = _layer128(x2, agg2, W2, b2.reshape(1, H), b3e.reshape(1, H))
    agg3 = _edge128(x3t, idxs, attrs, W3e[:, 0], z128)
    out = _final(x3, agg3, W3, b3.reshape(1, H),
                 batch.reshape(G, 1, BLK), Wo, bo.reshape(1, 5))
    return out


# speedup vs baseline: 15.6240x; 1.0013x over previous
import jax
import jax.numpy as jnp
from jax import lax
from jax.experimental import pallas as pl
from jax.experimental.pallas import tpu as pltpu
from jax.experimental.pallas import tpu_sc as plsc

N = 10000
E = 640000
NG = 64
H = 128
D1 = 16
NC = 2
NS = 16
NW = NC * NS
C = 96
K = 216
EW = C * K
E_PAD = EW * NW
N_PAD = 10240
RPT = N_PAD // NS
BLK = 2000
G = N // BLK


def _make_edge_phase(d, ck, kk, sub):
    mesh = plsc.VectorSubcoreMesh(core_axis_name="c", subcore_axis_name="s")
    sc = sub * ck

    def body(x_hbm, idx_hbm, attr_hbm, evec_hbm, zeros_hbm,
             out_hbm, ec_v, attr_c, rows_v, dstc_v, evec_v, agg_sh,
             gsem, ssem, isem):
        c = lax.axis_index("c")
        s = lax.axis_index("s")
        base = (c * NS + s) * kk
        pltpu.sync_copy(evec_hbm, evec_v)

        @pl.when(s == 0)
        def _():
            pltpu.sync_copy(zeros_hbm, agg_sh)

        plsc.subcore_barrier()

        ev = [evec_v[pl.ds(16 * j, 16)] for j in range(d // 16)]
        ec = tuple(ec_v.at[b] for b in range(3))
        att = tuple(attr_c.at[b] for b in range(3))
        rows = tuple(rows_v.at[b] for b in range(3))
        gs = tuple(gsem.at[b] for b in range(3))
        ss = tuple(ssem.at[b] for b in range(3))

        dc = tuple(dstc_v.at[b] for b in range(3))
        isem_t = tuple(isem.at[b] for b in range(3))

        def launch_idx(b, k):
            pltpu.async_copy(idx_hbm.at[base + k], ec[b], isem_t[b])
            pltpu.async_copy(attr_hbm.at[base + k], att[b], isem_t[b])

        def wait_idx(b, k):
            pltpu.make_async_copy(idx_hbm.at[base + k], ec[b],
                                  isem_t[b]).wait()
            pltpu.make_async_copy(attr_hbm.at[base + k], att[b],
                                  isem_t[b]).wait()

        def launch_gather(b):
            for m in range(sub):
                pltpu.async_copy(x_hbm.at[ec[b].at[0, m]],
                                 rows[b].at[pl.ds(m * ck, ck)], gs[b])

        def drain_gather(b):
            for m in range(sub):
                pltpu.make_async_copy(x_hbm.at[ec[b].at[0, m]],
                                      rows[b].at[pl.ds(m * ck, ck)],
                                      gs[b]).wait()

        def start_scatter(b):
            for m in range(sub):
                for w in range(ck // 16):
                    dc[b][m, pl.ds(16 * w, 16)] = \
                        ec[b][1, m, pl.ds(16 * w, 16)]
            for m in range(sub):
                pltpu.async_copy(rows[b].at[pl.ds(m * ck, ck)],
                                 agg_sh.at[dc[b].at[m]], ss[b], add=True)

        def drain_scatter(b):
            for m in range(sub):
                pltpu.make_async_copy(rows[b].at[pl.ds(m * ck, ck)],
                                      agg_sh.at[dc[b].at[m]],
                                      ss[b]).wait()

        for b in range(2):
            launch_idx(b, b)
        for b in range(2):
            wait_idx(b, b)
            launch_gather(b)

        def slot_body(i, carry):
            for b in range(3):
                k = 3 * i + b
                b2 = (b + 2) % 3

                @pl.when(k + 2 < kk)
                def _():
                    launch_idx(b2, k + 2)

                drain_gather(b)

                @plsc.parallel_loop(0, sc // 16, unroll=4)
                def group_body(g):
                    a16 = att[b][pl.ds(g * 16, 16)]
                    i0 = g * 16
                    for i2 in range(16):
                        a = a16[i2]
                        for j in range(d // 16):
                            sl = pl.ds(16 * j, 16)
                            rows[b][i0 + i2, sl] = jnp.maximum(
                                rows[b][i0 + i2, sl] + a * ev[j], 0.0)
                start_scatter(b)

                @pl.when(k >= 1)
                def _():
                    drain_scatter(b2)

                @pl.when(k + 2 < kk)
                def _():
                    wait_idx(b2, k + 2)
                    launch_gather(b2)
            return carry

        lax.fori_loop(0, kk // 3, slot_body, 0)
        drain_scatter((kk - 1) % 3)
        plsc.subcore_barrier()
        r0 = s * RPT
        pltpu.sync_copy(agg_sh.at[pl.ds(r0, RPT)],
                        out_hbm.at[c, pl.ds(r0, RPT)])

    return pl.kernel(
        body,
        out_type=jax.ShapeDtypeStruct((NC, N_PAD, d), jnp.float32),
        mesh=mesh,
        compiler_params=pltpu.CompilerParams(use_tc_tiling_on_sc=False),
        scratch_types=[
            pltpu.VMEM((3, 2, sub, ck), jnp.int32),
            pltpu.VMEM((3, sc), jnp.float32),
            pltpu.VMEM((3, sc, d), jnp.float32),
            pltpu.VMEM((3, sub, ck), jnp.int32),
            pltpu.VMEM((d,), jnp.float32),
            pltpu.VMEM_SHARED((N_PAD, d), jnp.float32),
            pltpu.SemaphoreType.DMA((3,)),
            pltpu.SemaphoreType.DMA((3,)),
            pltpu.SemaphoreType.DMA((3,)),
        ],
    )


C1 = 128
S1 = 6
K1 = 27
_edge16 = _make_edge_phase(D1, C1, K1, S1)
_edge128 = _make_edge_phase(H, C, K, 1)


def _tc_layer(din):
    def body(x_ref, agg_ref, w_ref, b_ref, ebn_ref, o_ref, ot_ref):
        h = x_ref[...] + agg_ref[0] + agg_ref[1]
        y = lax.dot_general(h, w_ref[...], (((1,), (1,)), ((), ())),
                            preferred_element_type=jnp.float32)
        y = jnp.maximum(y + b_ref[...], 0.0)
        o_ref[...] = y
        ot_ref[...] = y + ebn_ref[...]

    return pl.pallas_call(
        body,
        grid=(G,),
        in_specs=[
            pl.BlockSpec((BLK, din), lambda i: (i, 0)),
            pl.BlockSpec((NC, BLK, din), lambda i: (0, i, 0)),
            pl.BlockSpec((H, din), lambda i: (0, 0)),
            pl.BlockSpec((1, H), lambda i: (0, 0)),
            pl.BlockSpec((1, H), lambda i: (0, 0)),
        ],
        out_specs=[pl.BlockSpec((BLK, H), lambda i: (i, 0)),
                   pl.BlockSpec((BLK, H), lambda i: (i, 0))],
        out_shape=[jax.ShapeDtypeStruct((N, H), jnp.float32),
                   jax.ShapeDtypeStruct((N, H), jnp.float32)],
    )


_layer16 = _tc_layer(D1)
_layer128 = _tc_layer(H)


def _tc_final_body(x_ref, agg_ref, w_ref, b_ref, batch_ref, wo_ref, bo_ref,
                   o_ref, sums, counts):
    i = pl.program_id(0)

    @pl.when(i == 0)
    def _():
        sums[...] = jnp.zeros_like(sums)
        counts[...] = jnp.zeros_like(counts)

    h = x_ref[...] + agg_ref[0] + agg_ref[1]
    y = lax.dot_general(h, w_ref[...], (((1,), (1,)), ((), ())),
                        preferred_element_type=jnp.float32)
    y = jnp.maximum(y + b_ref[...], 0.0)
    bt = batch_ref[0]
    ohT = (lax.broadcasted_iota(jnp.int32, (NG, BLK), 0) == bt)
    ohT = ohT.astype(jnp.float32)
    sums[...] += lax.dot_general(ohT, y, (((1,), (0,)), ((), ())),
                                 preferred_element_type=jnp.float32)
    counts[...] += jnp.sum(ohT, axis=1, keepdims=True)

    @pl.when(i == G - 1)
    def _():
        pooled = sums[...] / jnp.maximum(counts[...], 1.0)
        o_ref[...] = lax.dot_general(pooled, wo_ref[...],
                                     (((1,), (1,)), ((), ())),
                                     preferred_element_type=jnp.float32) \
            + bo_ref[...]


_final = pl.pallas_call(
    _tc_final_body,
    grid=(G,),
    in_specs=[
        pl.BlockSpec((BLK, H), lambda i: (i, 0)),
        pl.BlockSpec((NC, BLK, H), lambda i: (0, i, 0)),
        pl.BlockSpec((H, H), lambda i: (0, 0)),
        pl.BlockSpec((1, H), lambda i: (0, 0)),
        pl.BlockSpec((1, 1, BLK), lambda i: (i, 0, 0)),
        pl.BlockSpec((5, H), lambda i: (0, 0)),
        pl.BlockSpec((1, 5), lambda i: (0, 0)),
    ],
    out_specs=pl.BlockSpec((NG, 5), lambda i: (0, 0)),
    out_shape=jax.ShapeDtypeStruct((NG, 5), jnp.float32),
    scratch_shapes=[
        pltpu.VMEM((NG, H), jnp.float32),
        pltpu.VMEM((NG, 1), jnp.float32),
    ],
)


def kernel(x, edge_index, edge_attr, batch,
           W1e, b1e, W1, b1, W2e, b2e, W2, b2, W3e, b3e, W3, b3, Wo, bo):
    src = edge_index[0]
    dst = edge_index[1]
    attr = edge_attr[:, 0]
    pad = E_PAD - E
    fill = jnp.arange(pad, dtype=jnp.int32)
    srcs = jnp.concatenate([src, fill % N])
    dsts = jnp.concatenate([dst, N + fill % (N_PAD - N)])
    attrs = jnp.concatenate([attr, jnp.zeros((pad,), jnp.float32)])
    idxs = jnp.stack([
        srcs.reshape(E_PAD // C, 1, C),
        dsts.reshape(E_PAD // C, 1, C),
    ], axis=1)
    sc1 = S1 * C1
    idxs1 = jnp.stack([
        srcs.reshape(E_PAD // sc1, S1, C1),
        dsts.reshape(E_PAD // sc1, S1, C1),
    ], axis=1)
    attrs1 = attrs.reshape(E_PAD // sc1, sc1)
    attrs = attrs.reshape(E_PAD // C, C)

    x1p = jnp.pad(x, ((0, 0), (0, D1 - 7)))
    w1p = jnp.pad(W1, ((0, 0), (0, D1 - 7)))
    ev1 = jnp.pad(W1e[:, 0], (0, D1 - 7))
    x1t = x1p + jnp.pad(b1e, (0, D1 - 7))[None, :]
    z16 = jnp.zeros((N_PAD, D1), jnp.float32)
    z128 = jnp.zeros((N_PAD, H), jnp.float32)

    agg1 = _edge16(x1t, idxs1, attrs1, ev1, z16)
    x2, x2t = _layer16(x1p, agg1, w1p, b1.reshape(1, H), b2e.reshape(1, H))
    agg2 = _edge128(x2t, idxs, attrs, W2e[:, 0], z128)
    x3, x3t = _layer128(x2, agg2, W2, b2.reshape(1, H), b3e.reshape(1, H))
    agg3 = _edge128(x3t, idxs, attrs, W3e[:, 0], z128)
    out = _final(x3, agg3, W3, b3.reshape(1, H),
                 batch.reshape(G, 1, BLK), Wo, bo.reshape(1, 5))
    return out


# speedup vs baseline: 16.0259x; 1.0257x over previous
import jax
import jax.numpy as jnp
from jax import lax
from jax.experimental import pallas as pl
from jax.experimental.pallas import tpu as pltpu
from jax.experimental.pallas import tpu_sc as plsc

N = 10000
E = 640000
NG = 64
H = 128
D1 = 16
NC = 2
NS = 16
NW = NC * NS
C = 112
K = 186
E_PAD2 = C * K * NW
E_PAD = 663552
N_PAD = 10240
RPT = N_PAD // NS
BLK = 2000
G = N // BLK


def _make_edge_phase(d, ck, kk, sub):
    mesh = plsc.VectorSubcoreMesh(core_axis_name="c", subcore_axis_name="s")
    sc = sub * ck

    def body(x_hbm, idx_hbm, attr_hbm, evec_hbm, zeros_hbm,
             out_hbm, ec_v, attr_c, rows_v, dstc_v, evec_v, agg_sh,
             gsem, ssem, isem):
        c = lax.axis_index("c")
        s = lax.axis_index("s")
        base = (c * NS + s) * kk
        pltpu.sync_copy(evec_hbm, evec_v)

        @pl.when(s == 0)
        def _():
            pltpu.sync_copy(zeros_hbm, agg_sh)

        plsc.subcore_barrier()

        ev = [evec_v[pl.ds(16 * j, 16)] for j in range(d // 16)]
        ec = tuple(ec_v.at[b] for b in range(3))
        att = tuple(attr_c.at[b] for b in range(3))
        rows = tuple(rows_v.at[b] for b in range(3))
        gs = tuple(gsem.at[b] for b in range(3))
        ss = tuple(ssem.at[b] for b in range(3))

        dc = tuple(dstc_v.at[b] for b in range(3))
        isem_t = tuple(isem.at[b] for b in range(3))

        def launch_idx(b, k):
            pltpu.async_copy(idx_hbm.at[base + k], ec[b], isem_t[b])
            pltpu.async_copy(attr_hbm.at[base + k], att[b], isem_t[b])

        def wait_idx(b, k):
            pltpu.make_async_copy(idx_hbm.at[base + k], ec[b],
                                  isem_t[b]).wait()
            pltpu.make_async_copy(attr_hbm.at[base + k], att[b],
                                  isem_t[b]).wait()

        def launch_gather(b):
            for m in range(sub):
                pltpu.async_copy(x_hbm.at[ec[b].at[0, m]],
                                 rows[b].at[pl.ds(m * ck, ck)], gs[b])

        def drain_gather(b):
            for m in range(sub):
                pltpu.make_async_copy(x_hbm.at[ec[b].at[0, m]],
                                      rows[b].at[pl.ds(m * ck, ck)],
                                      gs[b]).wait()

        def start_scatter(b):
            for m in range(sub):
                for w in range(ck // 16):
                    dc[b][m, pl.ds(16 * w, 16)] = \
                        ec[b][1, m, pl.ds(16 * w, 16)]
            for m in range(sub):
                pltpu.async_copy(rows[b].at[pl.ds(m * ck, ck)],
                                 agg_sh.at[dc[b].at[m]], ss[b], add=True)

        def drain_scatter(b):
            for m in range(sub):
                pltpu.make_async_copy(rows[b].at[pl.ds(m * ck, ck)],
                                      agg_sh.at[dc[b].at[m]],
                                      ss[b]).wait()

        for b in range(2):
            launch_idx(b, b)
        for b in range(2):
            wait_idx(b, b)
            launch_gather(b)

        def slot_body(i, carry):
            for b in range(3):
                k = 3 * i + b
                b2 = (b + 2) % 3

                @pl.when(k + 2 < kk)
                def _():
                    launch_idx(b2, k + 2)

                drain_gather(b)

                @plsc.parallel_loop(0, sc // 16, unroll=4)
                def group_body(g):
                    a16 = att[b][pl.ds(g * 16, 16)]
                    i0 = g * 16
                    for i2 in range(16):
                        a = a16[i2]
                        for j in range(d // 16):
                            sl = pl.ds(16 * j, 16)
                            rows[b][i0 + i2, sl] = jnp.maximum(
                                rows[b][i0 + i2, sl] + a * ev[j], 0.0)
                start_scatter(b)

                @pl.when(k >= 1)
                def _():
                    drain_scatter(b2)

                @pl.when(k + 2 < kk)
                def _():
                    wait_idx(b2, k + 2)
                    launch_gather(b2)
            return carry

        lax.fori_loop(0, kk // 3, slot_body, 0)
        drain_scatter((kk - 1) % 3)
        plsc.subcore_barrier()
        r0 = s * RPT
        pltpu.sync_copy(agg_sh.at[pl.ds(r0, RPT)],
                        out_hbm.at[c, pl.ds(r0, RPT)])

    return pl.kernel(
        body,
        out_type=jax.ShapeDtypeStruct((NC, N_PAD, d), jnp.float32),
        mesh=mesh,
        compiler_params=pltpu.CompilerParams(use_tc_tiling_on_sc=False),
        scratch_types=[
            pltpu.VMEM((3, 2, sub, ck), jnp.int32),
            pltpu.VMEM((3, sc), jnp.float32),
            pltpu.VMEM((3, sc, d), jnp.float32),
            pltpu.VMEM((3, sub, ck), jnp.int32),
            pltpu.VMEM((d,), jnp.float32),
            pltpu.VMEM_SHARED((N_PAD, d), jnp.float32),
            pltpu.SemaphoreType.DMA((3,)),
            pltpu.SemaphoreType.DMA((3,)),
            pltpu.SemaphoreType.DMA((3,)),
        ],
    )


C1 = 128
S1 = 6
K1 = 27
_edge16 = _make_edge_phase(D1, C1, K1, S1)
_edge128 = _make_edge_phase(H, C, K, 1)


def _tc_layer(din):
    def body(x_ref, agg_ref, w_ref, b_ref, ebn_ref, o_ref, ot_ref):
        h = x_ref[...] + agg_ref[0] + agg_ref[1]
        y = lax.dot_general(h, w_ref[...], (((1,), (1,)), ((), ())),
                            preferred_element_type=jnp.float32)
        y = jnp.maximum(y + b_ref[...], 0.0)
        o_ref[...] = y
        ot_ref[...] = y + ebn_ref[...]

    return pl.pallas_call(
        body,
        grid=(G,),
        in_specs=[
            pl.BlockSpec((BLK, din), lambda i: (i, 0)),
            pl.BlockSpec((NC, BLK, din), lambda i: (0, i, 0)),
            pl.BlockSpec((H, din), lambda i: (0, 0)),
            pl.BlockSpec((1, H), lambda i: (0, 0)),
            pl.BlockSpec((1, H), lambda i: (0, 0)),
        ],
        out_specs=[pl.BlockSpec((BLK, H), lambda i: (i, 0)),
                   pl.BlockSpec((BLK, H), lambda i: (i, 0))],
        out_shape=[jax.ShapeDtypeStruct((N, H), jnp.float32),
                   jax.ShapeDtypeStruct((N, H), jnp.float32)],
    )


_layer16 = _tc_layer(D1)
_layer128 = _tc_layer(H)


def _tc_final_body(x_ref, agg_ref, w_ref, b_ref, batch_ref, wo_ref, bo_ref,
                   o_ref, sums, counts):
    i = pl.program_id(0)

    @pl.when(i == 0)
    def _():
        sums[...] = jnp.zeros_like(sums)
        counts[...] = jnp.zeros_like(counts)

    h = x_ref[...] + agg_ref[0] + agg_ref[1]
    y = lax.dot_general(h, w_ref[...], (((1,), (1,)), ((), ())),
                        preferred_element_type=jnp.float32)
    y = jnp.maximum(y + b_ref[...], 0.0)
    bt = batch_ref[0]
    ohT = (lax.broadcasted_iota(jnp.int32, (NG, BLK), 0) == bt)
    ohT = ohT.astype(jnp.float32)
    sums[...] += lax.dot_general(ohT, y, (((1,), (0,)), ((), ())),
                                 preferred_element_type=jnp.float32)
    counts[...] += jnp.sum(ohT, axis=1, keepdims=True)

    @pl.when(i == G - 1)
    def _():
        pooled = sums[...] / jnp.maximum(counts[...], 1.0)
        o_ref[...] = lax.dot_general(pooled, wo_ref[...],
                                     (((1,), (1,)), ((), ())),
                                     preferred_element_type=jnp.float32) \
            + bo_ref[...]


_final = pl.pallas_call(
    _tc_final_body,
    grid=(G,),
    in_specs=[
        pl.BlockSpec((BLK, H), lambda i: (i, 0)),
        pl.BlockSpec((NC, BLK, H), lambda i: (0, i, 0)),
        pl.BlockSpec((H, H), lambda i: (0, 0)),
        pl.BlockSpec((1, H), lambda i: (0, 0)),
        pl.BlockSpec((1, 1, BLK), lambda i: (i, 0, 0)),
        pl.BlockSpec((5, H), lambda i: (0, 0)),
        pl.BlockSpec((1, 5), lambda i: (0, 0)),
    ],
    out_specs=pl.BlockSpec((NG, 5), lambda i: (0, 0)),
    out_shape=jax.ShapeDtypeStruct((NG, 5), jnp.float32),
    scratch_shapes=[
        pltpu.VMEM((NG, H), jnp.float32),
        pltpu.VMEM((NG, 1), jnp.float32),
    ],
)


def kernel(x, edge_index, edge_attr, batch,
           W1e, b1e, W1, b1, W2e, b2e, W2, b2, W3e, b3e, W3, b3, Wo, bo):
    src = edge_index[0]
    dst = edge_index[1]
    attr = edge_attr[:, 0]
    pad2 = E_PAD2 - E
    fill2 = jnp.arange(pad2, dtype=jnp.int32)
    srcs2 = jnp.concatenate([src, fill2 % N])
    dsts2 = jnp.concatenate([dst, N + fill2 % (N_PAD - N)])
    attrs2 = jnp.concatenate([attr, jnp.zeros((pad2,), jnp.float32)])
    idxs = jnp.stack([
        srcs2.reshape(E_PAD2 // C, 1, C),
        dsts2.reshape(E_PAD2 // C, 1, C),
    ], axis=1)
    attrs = attrs2.reshape(E_PAD2 // C, C)

    pad = E_PAD - E
    fill = jnp.arange(pad, dtype=jnp.int32)
    srcs = jnp.concatenate([src, fill % N])
    dsts = jnp.concatenate([dst, N + fill % (N_PAD - N)])
    attrs1 = jnp.concatenate([attr, jnp.zeros((pad,), jnp.float32)])
    sc1 = S1 * C1
    idxs1 = jnp.stack([
        srcs.reshape(E_PAD // sc1, S1, C1),
        dsts.reshape(E_PAD // sc1, S1, C1),
    ], axis=1)
    attrs1 = attrs1.reshape(E_PAD // sc1, sc1)

    x1p = jnp.pad(x, ((0, 0), (0, D1 - 7)))
    w1p = jnp.pad(W1, ((0, 0), (0, D1 - 7)))
    ev1 = jnp.pad(W1e[:, 0], (0, D1 - 7))
    x1t = x1p + jnp.pad(b1e, (0, D1 - 7))[None, :]
    z16 = jnp.zeros((N_PAD, D1), jnp.float32)
    z128 = jnp.zeros((N_PAD, H), jnp.float32)

    agg1 = _edge16(x1t, idxs1, attrs1, ev1, z16)
    x2, x2t = _layer16(x1p, agg1, w1p, b1.reshape(1, H), b2e.reshape(1, H))
    agg2 = _edge128(x2t, idxs, attrs, W2e[:, 0], z128)
    x3, x3t = _layer128(x2, agg2, W2, b2.reshape(1, H), b3e.reshape(1, H))
    agg3 = _edge128(x3t, idxs, attrs, W3e[:, 0], z128)
    out = _final(x3, agg3, W3, b3.reshape(1, H),
                 batch.reshape(G, 1, BLK), Wo, bo.reshape(1, 5))
    return out


# speedup vs baseline: 16.2145x; 1.0118x over previous
import jax
import jax.numpy as jnp
from jax import lax
from jax.experimental import pallas as pl
from jax.experimental.pallas import tpu as pltpu
from jax.experimental.pallas import tpu_sc as plsc

N = 10000
E = 640000
NG = 64
H = 128
D1 = 16
NC = 2
NS = 16
NW = NC * NS
C = 112
K = 186
E_PAD2 = C * K * NW
E_PAD = 663552
N_PAD = 10240
RPT = N_PAD // NS
BLK = 2000
G = N // BLK


def _make_edge_phase(d, ck, kk, sub):
    mesh = plsc.VectorSubcoreMesh(core_axis_name="c", subcore_axis_name="s")
    sc = sub * ck

    def body(x_hbm, idx_hbm, attr_hbm, evec_hbm, zeros_hbm,
             out_hbm, ec_v, attr_c, rows_v, dstc_v, evec_v, agg_sh,
             gsem, ssem, isem):
        c = lax.axis_index("c")
        s = lax.axis_index("s")
        base = (c * NS + s) * kk
        pltpu.sync_copy(evec_hbm, evec_v)
        ev = [evec_v[pl.ds(16 * j, 16)] for j in range(d // 16)]
        ec = tuple(ec_v.at[b] for b in range(3))
        att = tuple(attr_c.at[b] for b in range(3))
        rows = tuple(rows_v.at[b] for b in range(3))
        gs = tuple(gsem.at[b] for b in range(3))
        ss = tuple(ssem.at[b] for b in range(3))

        dc = tuple(dstc_v.at[b] for b in range(3))
        isem_t = tuple(isem.at[b] for b in range(3))

        def launch_idx(b, k):
            pltpu.async_copy(idx_hbm.at[base + k], ec[b], isem_t[b])
            pltpu.async_copy(attr_hbm.at[base + k], att[b], isem_t[b])

        def wait_idx(b, k):
            pltpu.make_async_copy(idx_hbm.at[base + k], ec[b],
                                  isem_t[b]).wait()
            pltpu.make_async_copy(attr_hbm.at[base + k], att[b],
                                  isem_t[b]).wait()

        def launch_gather(b):
            for m in range(sub):
                pltpu.async_copy(x_hbm.at[ec[b].at[0, m]],
                                 rows[b].at[pl.ds(m * ck, ck)], gs[b])

        def drain_gather(b):
            for m in range(sub):
                pltpu.make_async_copy(x_hbm.at[ec[b].at[0, m]],
                                      rows[b].at[pl.ds(m * ck, ck)],
                                      gs[b]).wait()

        def start_scatter(b):
            for m in range(sub):
                for w in range(ck // 16):
                    dc[b][m, pl.ds(16 * w, 16)] = \
                        ec[b][1, m, pl.ds(16 * w, 16)]
            for m in range(sub):
                pltpu.async_copy(rows[b].at[pl.ds(m * ck, ck)],
                                 agg_sh.at[dc[b].at[m]], ss[b], add=True)

        def drain_scatter(b):
            for m in range(sub):
                pltpu.make_async_copy(rows[b].at[pl.ds(m * ck, ck)],
                                      agg_sh.at[dc[b].at[m]],
                                      ss[b]).wait()

        for b in range(2):
            launch_idx(b, b)
        pltpu.sync_copy(zeros_hbm, agg_sh.at[pl.ds(s * RPT, RPT)])
        for b in range(2):
            wait_idx(b, b)
            launch_gather(b)
        plsc.subcore_barrier()

        def slot_body(i, carry):
            for b in range(3):
                k = 3 * i + b
                b2 = (b + 2) % 3

                @pl.when(k + 2 < kk)
                def _():
                    launch_idx(b2, k + 2)

                drain_gather(b)

                @plsc.parallel_loop(0, sc // 16, unroll=4)
                def group_body(g):
                    a16 = att[b][pl.ds(g * 16, 16)]
                    i0 = g * 16
                    for i2 in range(16):
                        a = a16[i2]
                        for j in range(d // 16):
                            sl = pl.ds(16 * j, 16)
                            rows[b][i0 + i2, sl] = jnp.maximum(
                                rows[b][i0 + i2, sl] + a * ev[j], 0.0)
                start_scatter(b)

                @pl.when(k >= 1)
                def _():
                    drain_scatter(b2)

                @pl.when(k + 2 < kk)
                def _():
                    wait_idx(b2, k + 2)
                    launch_gather(b2)
            return carry

        lax.fori_loop(0, kk // 3, slot_body, 0)
        drain_scatter((kk - 1) % 3)
        plsc.subcore_barrier()
        r0 = s * RPT
        pltpu.sync_copy(agg_sh.at[pl.ds(r0, RPT)],
                        out_hbm.at[c, pl.ds(r0, RPT)])

    return pl.kernel(
        body,
        out_type=jax.ShapeDtypeStruct((NC, N_PAD, d), jnp.float32),
        mesh=mesh,
        compiler_params=pltpu.CompilerParams(use_tc_tiling_on_sc=False),
        scratch_types=[
            pltpu.VMEM((3, 2, sub, ck), jnp.int32),
            pltpu.VMEM((3, sc), jnp.float32),
            pltpu.VMEM((3, sc, d), jnp.float32),
            pltpu.VMEM((3, sub, ck), jnp.int32),
            pltpu.VMEM((d,), jnp.float32),
            pltpu.VMEM_SHARED((N_PAD, d), jnp.float32),
            pltpu.SemaphoreType.DMA((3,)),
            pltpu.SemaphoreType.DMA((3,)),
            pltpu.SemaphoreType.DMA((3,)),
        ],
    )


C1 = 128
S1 = 6
K1 = 27
_edge16 = _make_edge_phase(D1, C1, K1, S1)
_edge128 = _make_edge_phase(H, C, K, 1)


def _tc_layer(din):
    def body(x_ref, agg_ref, w_ref, b_ref, ebn_ref, o_ref, ot_ref):
        h = x_ref[...] + agg_ref[0] + agg_ref[1]
        y = lax.dot_general(h, w_ref[...], (((1,), (1,)), ((), ())),
                            preferred_element_type=jnp.float32)
        y = jnp.maximum(y + b_ref[...], 0.0)
        o_ref[...] = y
        ot_ref[...] = y + ebn_ref[...]

    return pl.pallas_call(
        body,
        grid=(G,),
        in_specs=[
            pl.BlockSpec((BLK, din), lambda i: (i, 0)),
            pl.BlockSpec((NC, BLK, din), lambda i: (0, i, 0)),
            pl.BlockSpec((H, din), lambda i: (0, 0)),
            pl.BlockSpec((1, H), lambda i: (0, 0)),
            pl.BlockSpec((1, H), lambda i: (0, 0)),
        ],
        out_specs=[pl.BlockSpec((BLK, H), lambda i: (i, 0)),
                   pl.BlockSpec((BLK, H), lambda i: (i, 0))],
        out_shape=[jax.ShapeDtypeStruct((N, H), jnp.float32),
                   jax.ShapeDtypeStruct((N, H), jnp.float32)],
    )


_layer16 = _tc_layer(D1)
_layer128 = _tc_layer(H)


def _tc_final_body(x_ref, agg_ref, w_ref, b_ref, batch_ref, wo_ref, bo_ref,
                   o_ref, sums, counts):
    i = pl.program_id(0)

    @pl.when(i == 0)
    def _():
        sums[...] = jnp.zeros_like(sums)
        counts[...] = jnp.zeros_like(counts)

    h = x_ref[...] + agg_ref[0] + agg_ref[1]
    y = lax.dot_general(h, w_ref[...], (((1,), (1,)), ((), ())),
                        preferred_element_type=jnp.float32)
    y = jnp.maximum(y + b_ref[...], 0.0)
    bt = batch_ref[0]
    ohT = (lax.broadcasted_iota(jnp.int32, (NG, BLK), 0) == bt)
    ohT = ohT.astype(jnp.float32)
    sums[...] += lax.dot_general(ohT, y, (((1,), (0,)), ((), ())),
                                 preferred_element_type=jnp.float32)
    counts[...] += jnp.sum(ohT, axis=1, keepdims=True)

    @pl.when(i == G - 1)
    def _():
        pooled = sums[...] / jnp.maximum(counts[...], 1.0)
        o_ref[...] = lax.dot_general(pooled, wo_ref[...],
                                     (((1,), (1,)), ((), ())),
                                     preferred_element_type=jnp.float32) \
            + bo_ref[...]


_final = pl.pallas_call(
    _tc_final_body,
    grid=(G,),
    in_specs=[
        pl.BlockSpec((BLK, H), lambda i: (i, 0)),
        pl.BlockSpec((NC, BLK, H), lambda i: (0, i, 0)),
        pl.BlockSpec((H, H), lambda i: (0, 0)),
        pl.BlockSpec((1, H), lambda i: (0, 0)),
        pl.BlockSpec((1, 1, BLK), lambda i: (i, 0, 0)),
        pl.BlockSpec((5, H), lambda i: (0, 0)),
        pl.BlockSpec((1, 5), lambda i: (0, 0)),
    ],
    out_specs=pl.BlockSpec((NG, 5), lambda i: (0, 0)),
    out_shape=jax.ShapeDtypeStruct((NG, 5), jnp.float32),
    scratch_shapes=[
        pltpu.VMEM((NG, H), jnp.float32),
        pltpu.VMEM((NG, 1), jnp.float32),
    ],
)


def kernel(x, edge_index, edge_attr, batch,
           W1e, b1e, W1, b1, W2e, b2e, W2, b2, W3e, b3e, W3, b3, Wo, bo):
    src = edge_index[0]
    dst = edge_index[1]
    attr = edge_attr[:, 0]
    pad2 = E_PAD2 - E
    fill2 = jnp.arange(pad2, dtype=jnp.int32)
    srcs2 = jnp.concatenate([src, fill2 % N])
    dsts2 = jnp.concatenate([dst, N + fill2 % (N_PAD - N)])
    attrs2 = jnp.concatenate([attr, jnp.zeros((pad2,), jnp.float32)])
    idxs = jnp.stack([
        srcs2.reshape(E_PAD2 // C, 1, C),
        dsts2.reshape(E_PAD2 // C, 1, C),
    ], axis=1)
    attrs = attrs2.reshape(E_PAD2 // C, C)

    pad = E_PAD - E
    fill = jnp.arange(pad, dtype=jnp.int32)
    srcs = jnp.concatenate([src, fill % N])
    dsts = jnp.concatenate([dst, N + fill % (N_PAD - N)])
    attrs1 = jnp.concatenate([attr, jnp.zeros((pad,), jnp.float32)])
    sc1 = S1 * C1
    idxs1 = jnp.stack([
        srcs.reshape(E_PAD // sc1, S1, C1),
        dsts.reshape(E_PAD // sc1, S1, C1),
    ], axis=1)
    attrs1 = attrs1.reshape(E_PAD // sc1, sc1)

    x1p = jnp.pad(x, ((0, 0), (0, D1 - 7)))
    w1p = jnp.pad(W1, ((0, 0), (0, D1 - 7)))
    ev1 = jnp.pad(W1e[:, 0], (0, D1 - 7))
    x1t = x1p + jnp.pad(b1e, (0, D1 - 7))[None, :]
    z16 = jnp.zeros((RPT, D1), jnp.float32)
    z128 = jnp.zeros((RPT, H), jnp.float32)

    agg1 = _edge16(x1t, idxs1, attrs1, ev1, z16)
    x2, x2t = _layer16(x1p, agg1, w1p, b1.reshape(1, H), b2e.reshape(1, H))
    agg2 = _edge128(x2t, idxs, attrs, W2e[:, 0], z128)
    x3, x3t = _layer128(x2, agg2, W2, b2.reshape(1, H), b3e.reshape(1, H))
    agg3 = _edge128(x3t, idxs, attrs, W3e[:, 0], z128)
    out = _final(x3, agg3, W3, b3.reshape(1, H),
                 batch.reshape(G, 1, BLK), Wo, bo.reshape(1, 5))
    return out
